# baseline (speedup 1.0000x reference)
import jax
import jax.numpy as jnp
from jax.experimental import pallas as pl
from jax.experimental.pallas import tpu as pltpu

_NUM_ALT = 4
_D = 256
_B = 1024
_L = 200
_BB = 64


def _fused_body(idx_ref, gamma_ref, beta_ref, feat_ref, out_ref):
    idx = idx_ref[...]
    g = jnp.broadcast_to(gamma_ref[0, :][None, :], (_BB, _D))
    b = jnp.broadcast_to(beta_ref[0, :][None, :], (_BB, _D))
    for k in range(1, _NUM_ALT):
        sel = idx == k
        g = jnp.where(sel, gamma_ref[k, :][None, :], g)
        b = jnp.where(sel, beta_ref[k, :][None, :], b)
    out_ref[...] = feat_ref[...] * g[:, None, :] + b[:, None, :]


def kernel(feat, alt_idx, gamma, beta):
    idx2 = alt_idx.astype(jnp.int32).reshape(_B, 1)
    return pl.pallas_call(
        _fused_body,
        grid=(_B // _BB,),
        in_specs=[
            pl.BlockSpec((_BB, 1), lambda i: (i, 0)),
            pl.BlockSpec((_NUM_ALT, _D), lambda i: (0, 0)),
            pl.BlockSpec((_NUM_ALT, _D), lambda i: (0, 0)),
            pl.BlockSpec((_BB, _L, _D), lambda i: (i, 0, 0)),
        ],
        out_specs=pl.BlockSpec((_BB, _L, _D), lambda i: (i, 0, 0)),
        out_shape=jax.ShapeDtypeStruct((_B, _L, _D), jnp.float32),
        compiler_params=pltpu.CompilerParams(
            dimension_semantics=("parallel",),
            vmem_limit_bytes=112 * 1024 * 1024,
        ),
    )(idx2, gamma, beta, feat)


# speedup vs baseline: 1.0001x; 1.0001x over previous
import jax
import jax.numpy as jnp
from jax.experimental import pallas as pl
from jax.experimental.pallas import tpu as pltpu

_NUM_ALT = 4
_D = 256
_B = 1024
_L = 200
_BB = 64


def _fused_body(idx_ref, gamma_ref, beta_ref, feat_ref, out_ref):
    idx = idx_ref[...]
    g = jnp.broadcast_to(gamma_ref[0, :][None, :], (_BB, _D))
    b = jnp.broadcast_to(beta_ref[0, :][None, :], (_BB, _D))
    for k in range(1, _NUM_ALT):
        sel = idx == k
        g = jnp.where(sel, gamma_ref[k, :][None, :], g)
        b = jnp.where(sel, beta_ref[k, :][None, :], b)
    out_ref[...] = feat_ref[...] * g[:, None, :] + b[:, None, :]


def kernel(feat, alt_idx, gamma, beta):
    idx2 = alt_idx.astype(jnp.int32).reshape(_B, 1)
    return pl.pallas_call(
        _fused_body,
        grid=(_B // _BB,),
        in_specs=[
            pl.BlockSpec((_BB, 1), lambda i: (i, 0)),
            pl.BlockSpec((_NUM_ALT, _D), lambda i: (0, 0)),
            pl.BlockSpec((_NUM_ALT, _D), lambda i: (0, 0)),
            pl.BlockSpec((_BB, _L, _D), lambda i: (i, 0, 0)),
        ],
        out_specs=pl.BlockSpec((_BB, _L, _D), lambda i: (i, 0, 0)),
        out_shape=jax.ShapeDtypeStruct((_B, _L, _D), jnp.float32),
        compiler_params=pltpu.CompilerParams(
            dimension_semantics=("arbitrary",),
            vmem_limit_bytes=112 * 1024 * 1024,
        ),
    )(idx2, gamma, beta, feat)
